# baseline (speedup 1.0000x reference)
import jax
import jax.numpy as jnp
from jax.experimental import pallas as pl


def _add_pe_kernel(seq_ref, pe_ref, out_ref):
    out_ref[...] = seq_ref[...] + pe_ref[...]


def kernel(sequence, pe):
    B, S, D = sequence.shape
    BS = 2048
    while S % BS:
        BS //= 2
    nsb = S // BS
    flat = sequence.reshape(B * S, D)
    out = pl.pallas_call(
        _add_pe_kernel,
        grid=(nsb, B),
        in_specs=[
            pl.BlockSpec((BS, D), lambda i, j: (j * nsb + i, 0)),
            pl.BlockSpec((BS, D), lambda i, j: (i, 0)),
        ],
        out_specs=pl.BlockSpec((BS, D), lambda i, j: (j * nsb + i, 0)),
        out_shape=jax.ShapeDtypeStruct((B * S, D), sequence.dtype),
    )(flat, pe[:S])
    return out.reshape(B, S, D)
